# baseline (speedup 1.0000x reference)
import jax
import jax.numpy as jnp
from jax.experimental import pallas as pl
from jax.experimental.pallas import tpu as pltpu

_EPS = 1e-10
_MIN_EXP = -8
_NUM_EMB = 24
_OUT_D = 128
_EXP_D = 32
_LN10 = 2.302585092994046
_SQRT_LOG2E = 1.2011224087864498
_B = 256


def _body(xt_ref, rhs_ref, out_ref):
    x = xt_ref[...]
    s = x.shape[0]
    e = jnp.floor(jnp.log10(x + _EPS))
    m2 = (2.0 * _SQRT_LOG2E) * (x * jnp.exp(e * -_LN10))
    idx = jnp.clip(e.astype(jnp.int32) - _MIN_EXP, 0, _NUM_EMB - 1)
    mi = jax.lax.bitcast_convert_type(m2, jnp.int32)
    mh = jax.lax.bitcast_convert_type(
        jnp.bitwise_and(mi, jnp.int32(-65536)), jnp.float32)
    ml = m2 - mh
    rhs = rhs_ref[...]
    lanes24 = jax.lax.broadcasted_iota(jnp.int32, (s, _NUM_EMB), 1)
    lane128 = jax.lax.broadcasted_iota(jnp.int32, (1, _OUT_D), 1)
    ones2 = jnp.ones((s, 2), jnp.float32)
    for r in range(_B):
        ic = jax.lax.broadcast_in_dim(idx[:, r], (s, _NUM_EMB), (0,))
        ohf = (lanes24 == ic).astype(jnp.float32)
        lhs = jnp.concatenate(
            [ohf, mh[:, r:r + 1], ml[:, r:r + 1], ones2],
            axis=1).astype(jnp.bfloat16)
        d = jax.lax.dot_general(
            lhs, rhs, (((1,), (0,)), ((), ())),
            preferred_element_type=jnp.float32)
        out_ref[r] = jnp.where(lane128 < _EXP_D, d, jnp.exp2(-(d * d)))


@jax.jit
def kernel(numbers, table, q_values):
    b, s = numbers.shape
    xt = numbers.T
    qs = (2.0 * _SQRT_LOG2E) * q_values
    qhf = jax.lax.bitcast_convert_type(
        jnp.bitwise_and(jax.lax.bitcast_convert_type(qs, jnp.int32),
                        jnp.int32(-65536)), jnp.float32)
    qh = qhf.astype(jnp.bfloat16)
    ql = (qs - qhf).astype(jnp.bfloat16)
    rhs = jnp.zeros((_NUM_EMB + 4, _OUT_D), jnp.bfloat16)
    rhs = rhs.at[:_NUM_EMB, :_EXP_D].set(table.astype(jnp.bfloat16))
    rhs = rhs.at[_NUM_EMB, _EXP_D:].set(jnp.bfloat16(1.0))
    rhs = rhs.at[_NUM_EMB + 1, _EXP_D:].set(jnp.bfloat16(1.0))
    rhs = rhs.at[_NUM_EMB + 2, _EXP_D:].set(-qh)
    rhs = rhs.at[_NUM_EMB + 3, _EXP_D:].set(-ql)

    grid = (b // _B,)
    out = pl.pallas_call(
        _body,
        grid=grid,
        in_specs=[
            pl.BlockSpec((s, _B), lambda i: (0, i)),
            pl.BlockSpec((_NUM_EMB + 4, _OUT_D), lambda i: (0, 0)),
        ],
        out_specs=pl.BlockSpec((_B, s, _OUT_D), lambda i: (i, 0, 0)),
        out_shape=jax.ShapeDtypeStruct((b, s, _OUT_D), jnp.float32),
        compiler_params=pltpu.CompilerParams(
            dimension_semantics=("arbitrary",)),
    )(xt, rhs)
    return out


# speedup vs baseline: 1.0504x; 1.0504x over previous
import jax
import jax.numpy as jnp
from jax.experimental import pallas as pl
from jax.experimental.pallas import tpu as pltpu

_EPS = 1e-10
_MIN_EXP = -8
_NUM_EMB = 24
_OUT_D = 128
_EXP_D = 32
_LN10 = 2.302585092994046
_SQRT_LOG2E = 1.2011224087864498
_B = 512


def _body(xt_ref, rhs_ref, qs_ref, out_ref):
    x = xt_ref[...]
    s = x.shape[0]
    e = jnp.floor(jnp.log10(x + _EPS))
    m2 = (2.0 * _SQRT_LOG2E) * (x * jnp.exp(e * -_LN10))
    idx = jnp.clip(e.astype(jnp.int32) - _MIN_EXP, 0, _NUM_EMB - 1)
    rhs = rhs_ref[...]
    qs = qs_ref[...]
    lanes24 = jax.lax.broadcasted_iota(jnp.int32, (s, _NUM_EMB), 1)
    for r in range(_B):
        ic = jax.lax.broadcast_in_dim(idx[:, r], (s, _NUM_EMB), (0,))
        oh = (lanes24 == ic).astype(jnp.bfloat16)
        d = jax.lax.dot_general(
            oh, rhs, (((1,), (0,)), ((), ())),
            preferred_element_type=jnp.float32)
        mc = jax.lax.broadcast_in_dim(m2[:, r], (s, _OUT_D), (0,))
        t = mc - qs
        out_ref[r] = d + jnp.exp2(-(t * t))


@jax.jit
def kernel(numbers, table, q_values):
    b, s = numbers.shape
    xt = numbers.T
    rhs = jnp.zeros((_NUM_EMB, _OUT_D), jnp.bfloat16)
    rhs = rhs.at[:, :_EXP_D].set(table.astype(jnp.bfloat16))
    qs = jnp.concatenate(
        [jnp.full((_EXP_D,), 1e30, jnp.float32),
         (2.0 * _SQRT_LOG2E) * q_values]).reshape(1, _OUT_D)

    grid = (b // _B,)
    out = pl.pallas_call(
        _body,
        grid=grid,
        in_specs=[
            pl.BlockSpec((s, _B), lambda i: (0, i)),
            pl.BlockSpec((_NUM_EMB, _OUT_D), lambda i: (0, 0)),
            pl.BlockSpec((1, _OUT_D), lambda i: (0, 0)),
        ],
        out_specs=pl.BlockSpec((_B, s, _OUT_D), lambda i: (i, 0, 0)),
        out_shape=jax.ShapeDtypeStruct((b, s, _OUT_D), jnp.float32),
        compiler_params=pltpu.CompilerParams(
            dimension_semantics=("arbitrary",)),
    )(xt, rhs, qs)
    return out


# speedup vs baseline: 1.2623x; 1.2018x over previous
import jax
import jax.numpy as jnp
import numpy as np
from jax.experimental import pallas as pl
from jax.experimental.pallas import tpu as pltpu

_EPS = 1e-10
_MIN_EXP = -8
_NUM_EMB = 24
_OUT_D = 128
_EXP_D = 32
_SQRT_LOG2E = 1.2011224087864498
_B = 256


def _body(xt_ref, rhs_ref, plo_ref, pup_ref, qs_ref, out_ref):
    x = xt_ref[...]
    s = x.shape[0]
    y = x + _EPS
    rhs = rhs_ref[...]
    plo = plo_ref[...]
    pup = pup_ref[...]
    qs = qs_ref[...]
    lane128 = jax.lax.broadcasted_iota(jnp.int32, (1, _OUT_D), 1)
    for r in range(_B):
        yb = jax.lax.broadcast_in_dim(y[:, r], (s, _OUT_D), (0,))
        yb48 = yb[:, :2 * _NUM_EMB]
        oh = ((yb48 >= plo) & (yb48 < pup)).astype(jnp.bfloat16)
        d = jax.lax.dot_general(
            oh, rhs, (((1,), (0,)), ((), ())),
            preferred_element_type=jnp.float32)
        t = (yb - _EPS) * d - qs
        out_ref[r] = jnp.where(lane128 < _EXP_D, d, jnp.exp2(-(t * t)))


def _hilo(v):
    hi = jax.lax.bitcast_convert_type(
        jnp.bitwise_and(jax.lax.bitcast_convert_type(v, jnp.int32),
                        jnp.int32(-65536)), jnp.float32)
    return hi, v - hi


@jax.jit
def kernel(numbers, table, q_values):
    b, s = numbers.shape
    xt = numbers.T

    exps = np.arange(_MIN_EXP, _MIN_EXP + _NUM_EMB, dtype=np.float64)
    p10 = np.power(10.0, exps).astype(np.float32)
    lo_b = p10.copy()
    lo_b[0] = 0.0
    up_b = np.power(10.0, exps + 1.0).astype(np.float32)
    up_b[-1] = np.float32(np.inf)
    plo = jnp.asarray(np.tile(lo_b, 2).reshape(1, 2 * _NUM_EMB))
    pup = jnp.asarray(np.tile(up_b, 2).reshape(1, 2 * _NUM_EMB))

    rec = ((2.0 * _SQRT_LOG2E)
           * np.power(10.0, -exps)).astype(np.float32)
    rec_hi, rec_lo = _hilo(jnp.asarray(rec))
    tab_hi, tab_lo = _hilo(table)

    rhs = jnp.zeros((2 * _NUM_EMB, _OUT_D), jnp.bfloat16)
    rhs = rhs.at[:_NUM_EMB, :_EXP_D].set(tab_hi.astype(jnp.bfloat16))
    rhs = rhs.at[_NUM_EMB:, :_EXP_D].set(tab_lo.astype(jnp.bfloat16))
    rhs = rhs.at[:_NUM_EMB, _EXP_D:].set(
        rec_hi.astype(jnp.bfloat16)[:, None])
    rhs = rhs.at[_NUM_EMB:, _EXP_D:].set(
        rec_lo.astype(jnp.bfloat16)[:, None])

    qs = jnp.concatenate(
        [jnp.full((_EXP_D,), 1e30, jnp.float32),
         (2.0 * _SQRT_LOG2E) * q_values]).reshape(1, _OUT_D)

    grid = (b // _B,)
    out = pl.pallas_call(
        _body,
        grid=grid,
        in_specs=[
            pl.BlockSpec((s, _B), lambda i: (0, i)),
            pl.BlockSpec((2 * _NUM_EMB, _OUT_D), lambda i: (0, 0)),
            pl.BlockSpec((1, 2 * _NUM_EMB), lambda i: (0, 0)),
            pl.BlockSpec((1, 2 * _NUM_EMB), lambda i: (0, 0)),
            pl.BlockSpec((1, _OUT_D), lambda i: (0, 0)),
        ],
        out_specs=pl.BlockSpec((_B, s, _OUT_D), lambda i: (i, 0, 0)),
        out_shape=jax.ShapeDtypeStruct((b, s, _OUT_D), jnp.float32),
        compiler_params=pltpu.CompilerParams(
            dimension_semantics=("arbitrary",)),
    )(xt, rhs, plo, pup, qs)
    return out
